# Initial kernel scaffold; baseline (speedup 1.0000x reference)
import jax
import jax.numpy as jnp
from jax.experimental import pallas as pl


def kernel(sample, entity_embedding, relation_embedding):
    raise NotImplementedError("write your pallas kernel here")


# speedup vs baseline: 1.6651x; 1.6651x over previous
import functools

import jax
import jax.numpy as jnp
from jax import lax
from jax.experimental import pallas as pl
from jax.experimental.pallas import tpu as pltpu
from jax.experimental.pallas import tpu_sc as plsc

_GAMMA = 12.0
_B = 16384
_D = 128
_NC = 2
_NS = 16
_NW = _NC * _NS
_BPW = _B // _NW
_CH = 128
_NCHUNK = _BPW // _CH


def _tec_body(hid, rid, tid, ent, rel, out,
              idx_h, idx_r, idx_t, ob, h0, r0, t0, h1, r1, t1, sem0, sem1):
    wid = lax.axis_index("s") * _NC + lax.axis_index("c")

    pltpu.sync_copy(hid.at[wid], idx_h)
    pltpu.sync_copy(rid.at[wid], idx_r)
    pltpu.sync_copy(tid.at[wid], idx_t)

    bufs = ((h0, r0, t0, sem0), (h1, r1, t1, sem1))

    def start(c):
        hb, rb, tb, sem = bufs[c % 2]
        return (
            pltpu.async_copy(ent.at[idx_h.at[c]], hb, sem),
            pltpu.async_copy(rel.at[idx_r.at[c]], rb, sem),
            pltpu.async_copy(ent.at[idx_t.at[c]], tb, sem),
        )

    last_lane = lax.iota(jnp.int32, 16) == 15

    def compute(c):
        hb, rb, tb, _ = bufs[c % 2]

        def row_body(r, _):
            acc = jnp.zeros((16,), jnp.float32)
            for j in range(_D // 16):
                sl = pl.ds(j * 16, 16)
                h = hb[r, sl]
                rr = rb[r, sl]
                t = tb[r, sl]
                acc = acc + jnp.abs(h + rr - t)
            val = _GAMMA - plsc.cumsum(acc)
            idxv = jnp.full((16,), c * _CH + r, jnp.int32)
            plsc.store_scatter(ob, [idxv], val, mask=last_lane)
            return 0

        lax.fori_loop(0, _CH, row_body, 0)

    pending = start(0)
    for c in range(_NCHUNK):
        nxt = start(c + 1) if c + 1 < _NCHUNK else None
        for cp in pending:
            cp.wait()
        compute(c)
        pending = nxt

    pltpu.sync_copy(ob, out.at[pl.ds(wid * _BPW, _BPW)])


@functools.partial(
    pl.kernel,
    out_type=jax.ShapeDtypeStruct((_B,), jnp.float32),
    mesh=plsc.VectorSubcoreMesh(
        core_axis_name="c", subcore_axis_name="s",
        num_cores=_NC, num_subcores=_NS),
    compiler_params=pltpu.CompilerParams(needs_layout_passes=False),
    scratch_types=[
        pltpu.VMEM((_NCHUNK, _CH), jnp.int32),
        pltpu.VMEM((_NCHUNK, _CH), jnp.int32),
        pltpu.VMEM((_NCHUNK, _CH), jnp.int32),
        pltpu.VMEM((_BPW,), jnp.float32),
        pltpu.VMEM((_CH, _D), jnp.float32),
        pltpu.VMEM((_CH, _D), jnp.float32),
        pltpu.VMEM((_CH, _D), jnp.float32),
        pltpu.VMEM((_CH, _D), jnp.float32),
        pltpu.VMEM((_CH, _D), jnp.float32),
        pltpu.VMEM((_CH, _D), jnp.float32),
        pltpu.SemaphoreType.DMA,
        pltpu.SemaphoreType.DMA,
    ],
)
def _sc_score(hid, rid, tid, ent, rel, out, *scratch):
    _tec_body(hid, rid, tid, ent, rel, out, *scratch)


def kernel(sample, entity_embedding, relation_embedding):
    s = sample.astype(jnp.int32)
    hid = s[:, 0].reshape(_NW, _NCHUNK, _CH)
    rid = s[:, 1].reshape(_NW, _NCHUNK, _CH)
    tid = s[:, 2].reshape(_NW, _NCHUNK, _CH)
    out = _sc_score(hid, rid, tid, entity_embedding, relation_embedding)
    return out.reshape(_B, 1)
